# baseline (speedup 1.0000x reference)
import functools
import math

import jax
import jax.numpy as jnp
from jax import lax
from jax.experimental import pallas as pl
from jax.experimental.pallas import tpu as pltpu
from jax.experimental.pallas import tpu_sc as plsc

N = 10000
E = 320000
D = 128
H = 128
RATIO = 0.5
K1 = int(math.ceil(RATIO * N))
K2 = int(math.ceil(RATIO * K1))
K3 = int(math.ceil(RATIO * K2))

_NW = 32
_EPW = E // _NW


def _sc_mesh():
    return plsc.VectorSubcoreMesh(core_axis_name="c", subcore_axis_name="s")


def _edge_vals_call(dinv, h, mask, src, dst):
    @functools.partial(
        pl.kernel,
        out_type=jax.ShapeDtypeStruct((E,), jnp.float32),
        mesh=_sc_mesh(),
        compiler_params=pltpu.CompilerParams(needs_layout_passes=False),
        scratch_types=[
            pltpu.VMEM((N,), jnp.float32),
            pltpu.VMEM((N,), jnp.float32),
            pltpu.VMEM((N,), jnp.float32),
            pltpu.VMEM((_EPW,), jnp.int32),
            pltpu.VMEM((_EPW,), jnp.int32),
            pltpu.VMEM((_EPW,), jnp.float32),
        ],
    )
    def k(dinv_hbm, h_hbm, mask_hbm, src_hbm, dst_hbm, out_hbm,
          dinv_v, h_v, mask_v, src_v, dst_v, val_v):
        wid = lax.axis_index("s") * 2 + lax.axis_index("c")
        base = wid * _EPW
        pltpu.sync_copy(dinv_hbm, dinv_v)
        pltpu.sync_copy(h_hbm, h_v)
        pltpu.sync_copy(mask_hbm, mask_v)
        pltpu.sync_copy(src_hbm.at[pl.ds(base, _EPW)], src_v)
        pltpu.sync_copy(dst_hbm.at[pl.ds(base, _EPW)], dst_v)

        def body(i, _):
            sl = pl.ds(i * 16, 16)
            s = src_v[sl]
            d = dst_v[sl]
            a = plsc.load_gather(dinv_v, [s])
            b = plsc.load_gather(dinv_v, [d])
            ms = plsc.load_gather(mask_v, [s])
            md = plsc.load_gather(mask_v, [d])
            hh = plsc.load_gather(h_v, [s])
            val_v[sl] = ((a * b) * (ms * md)) * hh
            return 0

        lax.fori_loop(0, _EPW // 16, body, 0)
        pltpu.sync_copy(val_v, out_hbm.at[pl.ds(base, _EPW)])

    return k(dinv, h, mask, src, dst)


def _edge_mask_call(mask, src, dst):
    @functools.partial(
        pl.kernel,
        out_type=jax.ShapeDtypeStruct((E,), jnp.float32),
        mesh=_sc_mesh(),
        compiler_params=pltpu.CompilerParams(needs_layout_passes=False),
        scratch_types=[
            pltpu.VMEM((N,), jnp.float32),
            pltpu.VMEM((_EPW,), jnp.int32),
            pltpu.VMEM((_EPW,), jnp.int32),
            pltpu.VMEM((_EPW,), jnp.float32),
        ],
    )
    def k(mask_hbm, src_hbm, dst_hbm, out_hbm, mask_v, src_v, dst_v, w_v):
        wid = lax.axis_index("s") * 2 + lax.axis_index("c")
        base = wid * _EPW
        pltpu.sync_copy(mask_hbm, mask_v)
        pltpu.sync_copy(src_hbm.at[pl.ds(base, _EPW)], src_v)
        pltpu.sync_copy(dst_hbm.at[pl.ds(base, _EPW)], dst_v)

        def body(i, _):
            sl = pl.ds(i * 16, 16)
            ms = plsc.load_gather(mask_v, [src_v[sl]])
            md = plsc.load_gather(mask_v, [dst_v[sl]])
            w_v[sl] = ms * md
            return 0

        lax.fori_loop(0, _EPW // 16, body, 0)
        pltpu.sync_copy(w_v, out_hbm.at[pl.ds(base, _EPW)])

    return k(mask, src, dst)


def _argsort_desc_call(keys, M):
    TILES = 16
    VPT = M // TILES
    NV = VPT // 16
    L = M.bit_length() - 1
    S = L * (L + 1) // 2

    @functools.partial(
        pl.kernel,
        out_type=jax.ShapeDtypeStruct((M,), jnp.int32),
        mesh=_sc_mesh(),
        compiler_params=pltpu.CompilerParams(needs_layout_passes=False),
        scratch_types=[
            pltpu.VMEM((VPT,), jnp.float32),
            pltpu.VMEM((VPT,), jnp.int32),
            pltpu.VMEM((VPT,), jnp.float32),
            pltpu.VMEM((VPT,), jnp.int32),
            pltpu.VMEM_SHARED((M,), jnp.float32),
            pltpu.VMEM_SHARED((M,), jnp.int32),
        ],
    )
    def k(keys_hbm, out_hbm, kv, iv, pk, pi, ksh, ish):
        cid = lax.axis_index("c")
        t = lax.axis_index("s")

        @pl.when(cid == 0)
        def _():
            base = t * VPT
            pltpu.sync_copy(keys_hbm.at[pl.ds(base, VPT)], kv)
            lane = lax.iota(jnp.int32, (16,), 0)

            def init_body(v, _):
                iv[pl.ds(v * 16, 16)] = jnp.full((16,), base + v * 16,
                                                 jnp.int32) + lane
                return 0

            lax.fori_loop(0, NV, init_body, 0)

            def substage(carry):
                kk, d = carry

                def prec(a, ia, b, ib):
                    return (a > b) | ((a == b) & (ia < ib))

                def cross_tile():
                    pltpu.sync_copy(kv, ksh.at[pl.ds(base, VPT)])
                    pltpu.sync_copy(iv, ish.at[pl.ds(base, VPT)])
                    plsc.subcore_barrier()
                    pt = t ^ (d // VPT)
                    pbase = pt * VPT
                    pltpu.sync_copy(ksh.at[pl.ds(pbase, VPT)], pk)
                    pltpu.sync_copy(ish.at[pl.ds(pbase, VPT)], pi)
                    plsc.subcore_barrier()
                    up = ((base >> kk) & 1) == 0
                    flag = jnp.full((16,), ((t < pt) == up), jnp.bool_)

                    def body(v, _):
                        sl = pl.ds(v * 16, 16)
                        a, ia = kv[sl], iv[sl]
                        b, ib = pk[sl], pi[sl]
                        ta = prec(a, ia, b, ib) == flag
                        kv[sl] = jnp.where(ta, a, b)
                        iv[sl] = jnp.where(ta, ia, ib)
                        return 0

                    lax.fori_loop(0, NV, body, 0)

                def cross_vreg():
                    dd = d // 16
                    msk = dd - 1

                    def body(p, _):
                        v = ((p & ~msk) << 1) | (p & msk)
                        o1 = v * 16
                        o2 = o1 + d
                        up = (((base + o1) >> kk) & 1) == 0
                        flag = jnp.full((16,), up, jnp.bool_)
                        a, ia = kv[pl.ds(o1, 16)], iv[pl.ds(o1, 16)]
                        b, ib = kv[pl.ds(o2, 16)], iv[pl.ds(o2, 16)]
                        ta = prec(a, ia, b, ib) == flag
                        kv[pl.ds(o1, 16)] = jnp.where(ta, a, b)
                        iv[pl.ds(o1, 16)] = jnp.where(ta, ia, ib)
                        kv[pl.ds(o2, 16)] = jnp.where(ta, b, a)
                        iv[pl.ds(o2, 16)] = jnp.where(ta, ib, ia)
                        return 0

                    lax.fori_loop(0, NV // 2, body, 0)

                def intra_vreg():
                    jl = lane ^ d

                    def body(v, _):
                        sl = pl.ds(v * 16, 16)
                        a, ia = kv[sl], iv[sl]
                        b = a.at[jl].get(mode="promise_in_bounds")
                        ib = ia.at[jl].get(mode="promise_in_bounds")
                        ivec = jnp.full((16,), base + v * 16, jnp.int32) + lane
                        up = ((ivec >> kk) & 1) == 0
                        iltj = (ivec & d) == 0
                        ta = prec(a, ia, b, ib) == (iltj == up)
                        kv[sl] = jnp.where(ta, a, b)
                        iv[sl] = jnp.where(ta, ia, ib)
                        return 0

                    lax.fori_loop(0, NV, body, 0)

                lax.cond(d >= VPT, cross_tile,
                         lambda: lax.cond(d >= 16, cross_vreg, intra_vreg))
                return lax.cond(d > 1, lambda: (kk, d // 2),
                                lambda: (kk + 1, 1 << kk))

            lax.fori_loop(0, S, lambda s, c: substage(c), (1, 1))
            pltpu.sync_copy(iv, out_hbm.at[pl.ds(base, VPT)])

    return k(keys)


def _dense_relu_body(x_ref, w_ref, b_ref, o_ref):
    o_ref[...] = jnp.maximum(
        jnp.dot(x_ref[...], w_ref[...], preferred_element_type=jnp.float32)
        + b_ref[...],
        0.0,
    )


def _dense_relu(x, W, b, blk=2000):
    n, d = x.shape
    h = W.shape[1]
    grid = n // blk
    return pl.pallas_call(
        _dense_relu_body,
        grid=(grid,),
        in_specs=[
            pl.BlockSpec((blk, d), lambda i: (i, 0)),
            pl.BlockSpec((d, h), lambda i: (0, 0)),
            pl.BlockSpec((1, h), lambda i: (0, 0)),
        ],
        out_specs=pl.BlockSpec((blk, h), lambda i: (i, 0)),
        out_shape=jax.ShapeDtypeStruct((n, h), jnp.float32),
    )(x, W, b.reshape(1, h))


def _gcn_score(h, src, dst, mask_f, w=None):
    if w is None:
        w = _edge_mask_call(mask_f, src, dst)
    deg = jnp.zeros((N,), jnp.float32).at[dst].add(w) + 1.0
    dinv = 1.0 / jnp.sqrt(deg)
    val = _edge_vals_call(dinv, h, mask_f, src, dst)
    agg = jnp.zeros((N,), jnp.float32).at[dst].add(val)
    return agg + dinv * dinv * h


def _masked_readout(x, mask_f, k):
    neg = jnp.float32(-3.4e38)
    mx = jnp.max(jnp.where(mask_f[:, None] > 0, x, neg), axis=0)
    mn = jnp.sum(x * mask_f[:, None], axis=0) / jnp.float32(k)
    return jnp.concatenate([mx, mn])


def kernel(x, edge_index, batch, W1, b1, Ws1, bs1, W2, b2, Ws2, bs2, W3, b3,
           Ws3, bs3, L1W, L1b, L2W, L2b, L3W, L3b):
    src, dst = edge_index[0], edge_index[1]

    x1 = _dense_relu(x, W1, b1)
    ones = jnp.ones((N,), jnp.float32)
    s1 = _gcn_score((x1 @ Ws1)[:, 0], src, dst, ones,
                    w=jnp.ones((E,), jnp.float32)) + bs1[0]

    _, perm1 = jax.lax.top_k(s1, K1)
    m1 = jnp.zeros((N,), jnp.float32).at[perm1].set(1.0)
    xp1 = x1 * jnp.tanh(s1)[:, None]
    r1 = _masked_readout(xp1, m1, K1)

    x2h = _dense_relu(xp1, W2, b2)
    s2 = _gcn_score((x2h @ Ws2)[:, 0], src, dst, m1) + bs2[0]
    score2 = s2[perm1]

    _, p2loc = jax.lax.top_k(score2, K2)
    perm2 = perm1[p2loc]
    m2 = jnp.zeros((N,), jnp.float32).at[perm2].set(1.0)
    xp2 = x2h * jnp.tanh(s2)[:, None]
    r2 = _masked_readout(xp2, m2, K2)

    x3h = _dense_relu(xp2, W3, b3)
    s3 = _gcn_score((x3h @ Ws3)[:, 0], src, dst, m2) + bs3[0]
    score3 = s3[perm2]

    _, p3loc = jax.lax.top_k(score3, K3)
    perm3 = perm2[p3loc]
    m3 = jnp.zeros((N,), jnp.float32).at[perm3].set(1.0)
    xp3 = x3h * jnp.tanh(s3)[:, None]
    r3 = _masked_readout(xp3, m3, K3)

    xo = (r1 + r2 + r3)[None, :]
    v = jnp.maximum(xo @ L1W + L1b, 0.0)
    v = jnp.maximum(v @ L2W + L2b, 0.0)
    out = jax.nn.log_softmax(v @ L3W + L3b, axis=-1)

    return (out, out, s1, s1, score2, score2, score3, score3)
